# baseline (speedup 1.0000x reference)
import functools

import jax
import jax.numpy as jnp
from jax import lax
from jax.experimental import pallas as pl
from jax.experimental.pallas import tpu as pltpu
from jax.experimental.pallas import tpu_sc as plsc

NC = 2
NS = 16
NW = NC * NS

CHUNK = 4
SEG = 80

TBLK = 16384
TB4 = TBLK // 4


def _transpose_tc(W, V, D):
    NB = pl.cdiv(V, TBLK)

    def body(w_ref, o_ref):
        a = jnp.concatenate(
            [w_ref[:, m * TB4:(m + 1) * TB4] for m in range(4)], axis=0)
        o_ref[...] = lax.dot_general(
            a, jnp.eye(128, dtype=jnp.float32),
            dimension_numbers=(((0,), (0,)), ((), ())),
            preferred_element_type=jnp.float32)

    wt4 = pl.pallas_call(
        body,
        grid=(NB,),
        in_specs=[pl.BlockSpec((D, TBLK), lambda j: (0, j))],
        out_specs=pl.BlockSpec((TB4, 128), lambda j: (j, 0)),
        out_shape=jax.ShapeDtypeStruct((NB * TB4, 128), jnp.float32),
    )(W)
    return wt4.reshape(NB * TBLK, D)


def _remap(v):
    u = v & (TBLK - 1)
    return (v - u) + ((u & (TB4 - 1)) << 2) + (u >> (TB4.bit_length() - 1))


def _make_embed_bag(B, L, D):
    rows_per_w = B // NW
    n_chunks = rows_per_w // CHUNK
    nseg = (CHUNK * L) // SEG
    assert CHUNK * L == nseg * SEG
    mesh = plsc.VectorSubcoreMesh(core_axis_name="c", subcore_axis_name="s")

    @functools.partial(
        pl.kernel,
        out_type=jax.ShapeDtypeStruct((B, D), jnp.float32),
        mesh=mesh,
        compiler_params=pltpu.CompilerParams(use_tc_tiling_on_sc=False),
        scratch_types=[
            pltpu.VMEM((2, CHUNK * L), jnp.int32),
            pltpu.VMEM((2, CHUNK * L, D), jnp.float32),
            pltpu.VMEM((CHUNK, D), jnp.float32),
            pltpu.VMEM((D,), jnp.float32),
            pltpu.SemaphoreType.DMA,
            pltpu.SemaphoreType.DMA,
        ],
    )
    def embed_bag(tok_hbm, wt_hbm, b_hbm, out_hbm, idx_v, rows_v, outc_v,
                  bias_v, sem0, sem1):
        wid = lax.axis_index("s") * NC + lax.axis_index("c")
        base_row = wid * rows_per_w
        sems = (sem0, sem1)
        pltpu.sync_copy(b_hbm, bias_v)
        bias0 = bias_v[pl.ds(0, 16)]
        bias1 = bias_v[pl.ds(16, 16)]

        def fire(c, buf):
            row0 = base_row + c * CHUNK
            pltpu.sync_copy(tok_hbm.at[pl.ds(row0 * L, CHUNK * L)],
                            idx_v.at[buf])
            return [
                pltpu.async_copy(
                    wt_hbm.at[idx_v.at[buf].at[pl.ds(j * SEG, SEG)]],
                    rows_v.at[buf].at[pl.ds(j * SEG, SEG)],
                    sems[buf],
                )
                for j in range(nseg)
            ]

        cps = fire(0, 0)
        for c in range(n_chunks):
            buf = c % 2
            nxt = cps if c + 1 >= n_chunks else fire(c + 1, (c + 1) % 2)
            for cp in cps:
                cp.wait()
            cps = nxt
            for r in range(CHUNK):
                zero = jnp.zeros((16,), jnp.float32)

                def tok_body(l, acc):
                    a0, a1, a2, a3 = acc
                    t = r * L + 2 * l
                    a0 = a0 + rows_v[buf, t, pl.ds(0, 16)]
                    a1 = a1 + rows_v[buf, t, pl.ds(16, 16)]
                    a2 = a2 + rows_v[buf, t + 1, pl.ds(0, 16)]
                    a3 = a3 + rows_v[buf, t + 1, pl.ds(16, 16)]
                    return (a0, a1, a2, a3)

                a0, a1, a2, a3 = lax.fori_loop(
                    0, L // 2, tok_body, (bias0, bias1, zero, zero),
                    unroll=4)
                outc_v[r, pl.ds(0, 16)] = a0 + a2
                outc_v[r, pl.ds(16, 16)] = a1 + a3
            pltpu.sync_copy(outc_v,
                            out_hbm.at[pl.ds(base_row + c * CHUNK, CHUNK)])

    return embed_bag


def kernel(tokens, W, b):
    B, L = tokens.shape
    D, V = W.shape
    wt = _transpose_tc(W, V, D)
    out = _make_embed_bag(B, L, D)(_remap(tokens).reshape(-1), wt, b)
    return out[:, None, :]


# speedup vs baseline: 1.0119x; 1.0119x over previous
import functools

import jax
import jax.numpy as jnp
from jax import lax
from jax.experimental import pallas as pl
from jax.experimental.pallas import tpu as pltpu
from jax.experimental.pallas import tpu_sc as plsc

NC = 2
NS = 16
NW = NC * NS

CHUNK = 4
SEG = 200

TBLK = 16384
TB4 = TBLK // 4


def _transpose_tc(W, V, D):
    NB = pl.cdiv(V, TBLK)

    def body(w_ref, o_ref):
        a = jnp.concatenate(
            [w_ref[:, m * TB4:(m + 1) * TB4] for m in range(4)], axis=0)
        o_ref[...] = lax.dot_general(
            a, jnp.eye(128, dtype=jnp.float32),
            dimension_numbers=(((0,), (0,)), ((), ())),
            preferred_element_type=jnp.float32)

    wt4 = pl.pallas_call(
        body,
        grid=(NB,),
        in_specs=[pl.BlockSpec((D, TBLK), lambda j: (0, j))],
        out_specs=pl.BlockSpec((TB4, 128), lambda j: (j, 0)),
        out_shape=jax.ShapeDtypeStruct((NB * TB4, 128), jnp.float32),
    )(W)
    return wt4.reshape(NB * TBLK, D)


def _remap(v):
    u = v & (TBLK - 1)
    return (v - u) + ((u & (TB4 - 1)) << 2) + (u >> (TB4.bit_length() - 1))


def _make_embed_bag(B, L, D):
    rows_per_w = B // NW
    n_chunks = rows_per_w // CHUNK
    nseg = (CHUNK * L) // SEG
    assert CHUNK * L == nseg * SEG
    mesh = plsc.VectorSubcoreMesh(core_axis_name="c", subcore_axis_name="s")

    @functools.partial(
        pl.kernel,
        out_type=jax.ShapeDtypeStruct((B, D), jnp.float32),
        mesh=mesh,
        compiler_params=pltpu.CompilerParams(use_tc_tiling_on_sc=False),
        scratch_types=[
            pltpu.VMEM((2, CHUNK * L), jnp.int32),
            pltpu.VMEM((2, CHUNK * L, D), jnp.float32),
            pltpu.VMEM((CHUNK, D), jnp.float32),
            pltpu.VMEM((D,), jnp.float32),
            pltpu.SemaphoreType.DMA,
            pltpu.SemaphoreType.DMA,
        ],
    )
    def embed_bag(tok_hbm, wt_hbm, b_hbm, out_hbm, idx_v, rows_v, outc_v,
                  bias_v, sem0, sem1):
        wid = lax.axis_index("s") * NC + lax.axis_index("c")
        base_row = wid * rows_per_w
        sems = (sem0, sem1)
        pltpu.sync_copy(b_hbm, bias_v)
        bias0 = bias_v[pl.ds(0, 16)]
        bias1 = bias_v[pl.ds(16, 16)]

        def fire(c, buf):
            row0 = base_row + c * CHUNK
            pltpu.sync_copy(tok_hbm.at[pl.ds(row0 * L, CHUNK * L)],
                            idx_v.at[buf])
            return [
                pltpu.async_copy(
                    wt_hbm.at[idx_v.at[buf].at[pl.ds(j * SEG, SEG)]],
                    rows_v.at[buf].at[pl.ds(j * SEG, SEG)],
                    sems[buf],
                )
                for j in range(nseg)
            ]

        cps = fire(0, 0)
        for c in range(n_chunks):
            buf = c % 2
            nxt = cps if c + 1 >= n_chunks else fire(c + 1, (c + 1) % 2)
            for cp in cps:
                cp.wait()
            cps = nxt
            for r in range(CHUNK):
                def tok_body(l, acc):
                    a0, a1 = acc
                    a0 = a0 + rows_v[buf, r * L + l, pl.ds(0, 16)]
                    a1 = a1 + rows_v[buf, r * L + l, pl.ds(16, 16)]
                    return (a0, a1)

                a0, a1 = lax.fori_loop(0, L, tok_body, (bias0, bias1),
                                       unroll=8)
                outc_v[r, pl.ds(0, 16)] = a0
                outc_v[r, pl.ds(16, 16)] = a1
            pltpu.sync_copy(outc_v,
                            out_hbm.at[pl.ds(base_row + c * CHUNK, CHUNK)])

    return embed_bag


def kernel(tokens, W, b):
    B, L = tokens.shape
    D, V = W.shape
    wt = _transpose_tc(W, V, D)
    out = _make_embed_bag(B, L, D)(_remap(tokens).reshape(-1), wt, b)
    return out[:, None, :]


# speedup vs baseline: 1.0297x; 1.0176x over previous
import functools

import jax
import jax.numpy as jnp
from jax import lax
from jax.experimental import pallas as pl
from jax.experimental.pallas import tpu as pltpu
from jax.experimental.pallas import tpu_sc as plsc

NC = 2
NS = 16
NW = NC * NS

CHUNK = 4
SEG = 200

TBLK = 16384
TB4 = TBLK // 4


def _transpose_tc(W, V, D):
    NB = pl.cdiv(V, TBLK)

    def body(w_ref, o_ref):
        a = jnp.concatenate(
            [w_ref[:, m * TB4:(m + 1) * TB4] for m in range(4)], axis=0)
        o_ref[...] = lax.dot_general(
            a, jnp.eye(128, dtype=jnp.float32),
            dimension_numbers=(((0,), (0,)), ((), ())),
            preferred_element_type=jnp.float32)

    wt4 = pl.pallas_call(
        body,
        grid=(NB,),
        in_specs=[pl.BlockSpec((D, TBLK), lambda j: (0, j))],
        out_specs=pl.BlockSpec((TB4, 128), lambda j: (j, 0)),
        out_shape=jax.ShapeDtypeStruct((NB * TB4, 128), jnp.float32),
    )(W)
    return wt4.reshape(NB * TBLK, D)


def _remap(v):
    u = v & (TBLK - 1)
    return (v - u) + ((u & (TB4 - 1)) << 2) + (u >> (TB4.bit_length() - 1))


def _make_embed_bag(B, L, D):
    rows_per_w = B // NW
    n_chunks = rows_per_w // CHUNK
    nseg = (CHUNK * L) // SEG
    assert CHUNK * L == nseg * SEG
    mesh = plsc.VectorSubcoreMesh(core_axis_name="c", subcore_axis_name="s")

    @functools.partial(
        pl.kernel,
        out_type=jax.ShapeDtypeStruct((B, D), jnp.float32),
        mesh=mesh,
        compiler_params=pltpu.CompilerParams(use_tc_tiling_on_sc=False),
        scratch_types=[
            pltpu.VMEM((2, CHUNK * L), jnp.int32),
            pltpu.VMEM((2, CHUNK * L, D), jnp.float32),
            pltpu.VMEM((CHUNK, D), jnp.float32),
            pltpu.VMEM((D,), jnp.float32),
            pltpu.SemaphoreType.DMA,
            pltpu.SemaphoreType.DMA,
        ],
    )
    def embed_bag(tok_hbm, wt_hbm, b_hbm, out_hbm, idx_v, rows_v, outc_v,
                  bias_v, sem0, sem1):
        wid = lax.axis_index("s") * NC + lax.axis_index("c")
        base_row = wid * rows_per_w
        sems = (sem0, sem1)

        def fire(c, buf):
            row0 = base_row + c * CHUNK
            pltpu.sync_copy(tok_hbm.at[pl.ds(row0 * L, CHUNK * L)],
                            idx_v.at[buf])
            return [
                pltpu.async_copy(
                    wt_hbm.at[idx_v.at[buf].at[pl.ds(j * SEG, SEG)]],
                    rows_v.at[buf].at[pl.ds(j * SEG, SEG)],
                    sems[buf],
                )
                for j in range(nseg)
            ]

        cps = fire(0, 0)
        pltpu.sync_copy(b_hbm, bias_v)
        bias0 = bias_v[pl.ds(0, 16)]
        bias1 = bias_v[pl.ds(16, 16)]
        for c in range(n_chunks):
            buf = c % 2
            nxt = cps if c + 1 >= n_chunks else fire(c + 1, (c + 1) % 2)
            for cp in cps:
                cp.wait()
            cps = nxt
            for r in range(CHUNK):
                def tok_body(l, acc):
                    a0, a1 = acc
                    a0 = a0 + rows_v[buf, r * L + l, pl.ds(0, 16)]
                    a1 = a1 + rows_v[buf, r * L + l, pl.ds(16, 16)]
                    return (a0, a1)

                a0, a1 = lax.fori_loop(0, L, tok_body, (bias0, bias1),
                                       unroll=8)
                outc_v[r, pl.ds(0, 16)] = a0
                outc_v[r, pl.ds(16, 16)] = a1
            pltpu.sync_copy(outc_v,
                            out_hbm.at[pl.ds(base_row + c * CHUNK, CHUNK)])

    return embed_bag


def kernel(tokens, W, b):
    B, L = tokens.shape
    D, V = W.shape
    wt = _transpose_tc(W, V, D)
    out = _make_embed_bag(B, L, D)(_remap(tokens).reshape(-1), wt, b)
    return out[:, None, :]
